# Initial kernel scaffold; baseline (speedup 1.0000x reference)
import jax
import jax.numpy as jnp
from jax.experimental import pallas as pl


def kernel(x, edge_index, edge_attr, W0, b0, W1, b1):
    raise NotImplementedError("write your pallas kernel here")


# speedup vs baseline: 11.6852x; 11.6852x over previous
import functools

import jax
import jax.numpy as jnp
from jax import lax
from jax.experimental import pallas as pl
from jax.experimental.pallas import tpu as pltpu
from jax.experimental.pallas import tpu_sc as plsc

NN = 10000
EE = 320000
DD = 128

NC = 2
NS = 16
NW = NC * NS
EPT = EE // NW
CH = 80
NCHUNK = EPT // CH
NPAD = 10240
DSL = NPAD // NS
ZSL = NN // NS
WR = 632
WR_LAST = NN - (NS - 1) * WR

_MESH = plsc.VectorSubcoreMesh(core_axis_name="c", subcore_axis_name="s")
_SC_PARAMS = pltpu.CompilerParams(needs_layout_passes=False)


@functools.partial(
    pl.kernel,
    out_type=jax.ShapeDtypeStruct((NC * NPAD,), jnp.float32),
    mesh=_MESH,
    compiler_params=_SC_PARAMS,
    scratch_types=[
        pltpu.VMEM((NCHUNK, CH), jnp.int32),
        pltpu.VMEM((NCHUNK, CH), jnp.float32),
        pltpu.VMEM((DSL,), jnp.float32),
        pltpu.VMEM_SHARED((NPAD,), jnp.float32),
    ],
)
def _deg_pass(dst_hbm, w_hbm, out_hbm, dst_v, w_v, stage_v, deg_sh):
    c = lax.axis_index("c")
    s = lax.axis_index("s")
    wid = c * NS + s
    pltpu.sync_copy(dst_hbm.at[wid], dst_v)
    pltpu.sync_copy(w_hbm.at[wid], w_v)

    def _zero(i, carry):
        stage_v[pl.ds(i * 16, 16)] = jnp.zeros((16,), jnp.float32)
        return carry

    lax.fori_loop(0, DSL // 16, _zero, 0)
    pltpu.sync_copy(stage_v, deg_sh.at[pl.ds(s * DSL, DSL)])
    plsc.subcore_barrier()

    def _scatter(j, carry):
        pltpu.sync_copy(w_v.at[j], deg_sh.at[dst_v.at[j]], add=True)
        return carry

    lax.fori_loop(0, NCHUNK, _scatter, 0)
    plsc.subcore_barrier()
    pltpu.sync_copy(deg_sh.at[pl.ds(s * DSL, DSL)],
                    out_hbm.at[pl.ds(c * NPAD + s * DSL, DSL)])


@functools.partial(
    pl.kernel,
    out_type=jax.ShapeDtypeStruct((NC, NN, DD), jnp.float32),
    mesh=_MESH,
    compiler_params=_SC_PARAMS,
    scratch_types=[
        pltpu.VMEM((EPT,), jnp.int32),
        pltpu.VMEM((NCHUNK, CH), jnp.int32),
        pltpu.VMEM((EPT,), jnp.float32),
        pltpu.VMEM((CH, DD), jnp.float32),
        pltpu.VMEM_SHARED((NN, DD), jnp.float32),
        pltpu.SemaphoreType.DMA,
    ],
)
def _edge_pass(hp_hbm, srcf_hbm, dst_hbm, wf_hbm, out_hbm,
               src_v, dst_v, w_v, rows_v, acc_sh, sem):
    c = lax.axis_index("c")
    s = lax.axis_index("s")
    wid = c * NS + s
    pltpu.sync_copy(srcf_hbm.at[pl.ds(wid * EPT, EPT)], src_v)
    pltpu.sync_copy(dst_hbm.at[wid], dst_v)
    pltpu.sync_copy(wf_hbm.at[pl.ds(wid * EPT, EPT)], w_v)

    def _zero(r, carry):
        for k in range(DD // 16):
            rows_v[r, pl.ds(k * 16, 16)] = jnp.zeros((16,), jnp.float32)
        return carry

    lax.fori_loop(0, CH, _zero, 0)
    for t in range(ZSL // CH):
        pltpu.sync_copy(rows_v, acc_sh.at[pl.ds(s * ZSL + t * CH, CH)])
    pltpu.sync_copy(rows_v.at[pl.ds(0, ZSL % CH)],
                    acc_sh.at[pl.ds(s * ZSL + (ZSL // CH) * CH, ZSL % CH)])
    plsc.subcore_barrier()

    def _chunk(j, carry):
        pltpu.async_copy(hp_hbm.at[src_v.at[pl.ds(j * CH, CH)]],
                         rows_v, sem).wait()

        def _scale(i, c2):
            wb = plsc.load_gather(w_v, [jnp.full((16,), j * CH + i, jnp.int32)])
            for k in range(DD // 16):
                rows_v[i, pl.ds(k * 16, 16)] = rows_v[i, pl.ds(k * 16, 16)] * wb
            return c2

        lax.fori_loop(0, CH, _scale, 0)
        pltpu.sync_copy(rows_v, acc_sh.at[dst_v.at[j]], add=True)
        return carry

    lax.fori_loop(0, NCHUNK, _chunk, 0)
    plsc.subcore_barrier()

    @pl.when(s < NS - 1)
    def _():
        pltpu.sync_copy(acc_sh.at[pl.ds(s * WR, WR)],
                        out_hbm.at[c, pl.ds(s * WR, WR)])

    @pl.when(s == NS - 1)
    def _():
        pltpu.sync_copy(acc_sh.at[pl.ds((NS - 1) * WR, WR_LAST)],
                        out_hbm.at[c, pl.ds((NS - 1) * WR, WR_LAST)])


BN = 1000


def _dinv_of(deg_blk):
    dsum = deg_blk[0] + deg_blk[1]
    safe = jnp.where(dsum > 0, dsum, 1.0)
    return jnp.where(dsum > 0, lax.rsqrt(safe), 0.0)


def _tc_a_body(deg_ref, x_ref, w_ref, hp_ref):
    dinv = _dinv_of(deg_ref[...])
    h = jnp.dot(x_ref[...], w_ref[...], preferred_element_type=jnp.float32)
    hp_ref[...] = h * dinv


def _tc_b_body(deg_ref, acc_ref, b_ref, w_ref, h0r_ref, h1p_ref):
    dinv = _dinv_of(deg_ref[...])
    a = acc_ref[0] + acc_ref[1]
    h0r = jnp.maximum(a * dinv + b_ref[...], 0.0)
    h0r_ref[...] = h0r
    h1p_ref[...] = jnp.dot(h0r, w_ref[...],
                           preferred_element_type=jnp.float32) * dinv


def _tc_c_body(deg_ref, acc_ref, b_ref, h_ref):
    dinv = _dinv_of(deg_ref[...])
    h_ref[...] = (acc_ref[0] + acc_ref[1]) * dinv + b_ref[...]


_deg_spec = pl.BlockSpec((NC, BN, 1), lambda i: (0, i, 0))
_row_spec = pl.BlockSpec((BN, DD), lambda i: (i, 0))
_acc_spec = pl.BlockSpec((NC, BN, DD), lambda i: (0, i, 0))
_mat_spec = pl.BlockSpec((DD, DD), lambda i: (0, 0))
_bias_spec = pl.BlockSpec((1, DD), lambda i: (0, 0))
_GRID = NN // BN

_tc_a = pl.pallas_call(
    _tc_a_body,
    grid=(_GRID,),
    in_specs=[_deg_spec, _row_spec, _mat_spec],
    out_specs=_row_spec,
    out_shape=jax.ShapeDtypeStruct((NN, DD), jnp.float32),
)

_tc_b = pl.pallas_call(
    _tc_b_body,
    grid=(_GRID,),
    in_specs=[_deg_spec, _acc_spec, _bias_spec, _mat_spec],
    out_specs=[_row_spec, _row_spec],
    out_shape=[jax.ShapeDtypeStruct((NN, DD), jnp.float32),
               jax.ShapeDtypeStruct((NN, DD), jnp.float32)],
)

_tc_c = pl.pallas_call(
    _tc_c_body,
    grid=(_GRID,),
    in_specs=[_deg_spec, _acc_spec, _bias_spec],
    out_specs=_row_spec,
    out_shape=jax.ShapeDtypeStruct((NN, DD), jnp.float32),
)


def kernel(x, edge_index, edge_attr, W0, b0, W1, b1):
    srcf = edge_index[0]
    dst3 = edge_index[1].reshape(NW, NCHUNK, CH)
    w3 = edge_attr.reshape(NW, NCHUNK, CH)

    deg_flat = _deg_pass(dst3, w3)
    deg2col = deg_flat.reshape(NC, NPAD)[:, :NN].reshape(NC, NN, 1)

    h0p = _tc_a(deg2col, x, W0)
    acc0 = _edge_pass(h0p, srcf, dst3, edge_attr)
    h0r, h1p = _tc_b(deg2col, acc0, b0.reshape(1, DD), W1)
    acc1 = _edge_pass(h1p, srcf, dst3, edge_attr)
    h1 = _tc_c(deg2col, acc1, b1.reshape(1, DD))
    return (h0r, h1)


# speedup vs baseline: 17.5529x; 1.5022x over previous
import functools

import jax
import jax.numpy as jnp
from jax import lax
from jax.experimental import pallas as pl
from jax.experimental.pallas import tpu as pltpu
from jax.experimental.pallas import tpu_sc as plsc

NN = 10000
EE = 320000
DD = 128

NC = 2
NS = 16
NW = NC * NS
EPT = EE // NW
CH = 80
NCHUNK = EPT // CH
NPAD = 10240
DSL = NPAD // NS
ZSL = NN // NS
WR = 632
WR_LAST = NN - (NS - 1) * WR

_MESH = plsc.VectorSubcoreMesh(core_axis_name="c", subcore_axis_name="s")
_SC_PARAMS = pltpu.CompilerParams(needs_layout_passes=False)


@functools.partial(
    pl.kernel,
    out_type=jax.ShapeDtypeStruct((NC * NPAD,), jnp.float32),
    mesh=_MESH,
    compiler_params=_SC_PARAMS,
    scratch_types=[
        pltpu.VMEM((NCHUNK, CH), jnp.int32),
        pltpu.VMEM((NCHUNK, CH), jnp.float32),
        pltpu.VMEM((DSL,), jnp.float32),
        pltpu.VMEM_SHARED((NPAD,), jnp.float32),
    ],
)
def _deg_pass(dst_hbm, w_hbm, out_hbm, dst_v, w_v, stage_v, deg_sh):
    c = lax.axis_index("c")
    s = lax.axis_index("s")
    wid = c * NS + s
    pltpu.sync_copy(dst_hbm.at[wid], dst_v)
    pltpu.sync_copy(w_hbm.at[wid], w_v)

    def _zero(i, carry):
        stage_v[pl.ds(i * 16, 16)] = jnp.zeros((16,), jnp.float32)
        return carry

    lax.fori_loop(0, DSL // 16, _zero, 0)
    pltpu.sync_copy(stage_v, deg_sh.at[pl.ds(s * DSL, DSL)])
    plsc.subcore_barrier()

    def _scatter(j, carry):
        pltpu.sync_copy(w_v.at[j], deg_sh.at[dst_v.at[j]], add=True)
        return carry

    lax.fori_loop(0, NCHUNK, _scatter, 0)
    plsc.subcore_barrier()
    pltpu.sync_copy(deg_sh.at[pl.ds(s * DSL, DSL)],
                    out_hbm.at[pl.ds(c * NPAD + s * DSL, DSL)])


@functools.partial(
    pl.kernel,
    out_type=jax.ShapeDtypeStruct((NC, NN, DD), jnp.float32),
    mesh=_MESH,
    compiler_params=_SC_PARAMS,
    scratch_types=[
        pltpu.VMEM((EPT,), jnp.int32),
        pltpu.VMEM((EPT,), jnp.int32),
        pltpu.VMEM((EPT,), jnp.float32),
        pltpu.VMEM((CH, DD), jnp.float32),
        pltpu.VMEM((CH, DD), jnp.float32),
        pltpu.SemaphoreType.DMA,
        pltpu.SemaphoreType.DMA,
        pltpu.SemaphoreType.DMA,
        pltpu.SemaphoreType.DMA,
        pltpu.VMEM_SHARED((NN, DD), jnp.float32),
    ],
)
def _edge_pass(hp_hbm, srcf_hbm, dstf_hbm, wf_hbm, out_hbm,
               src_v, dst_v, w_v, rows0, rows1, g0, g1, s0, s1, acc_sh):
    c = lax.axis_index("c")
    s = lax.axis_index("s")
    wid = c * NS + s
    pltpu.sync_copy(srcf_hbm.at[pl.ds(wid * EPT, EPT)], src_v)
    pltpu.sync_copy(dstf_hbm.at[pl.ds(wid * EPT, EPT)], dst_v)
    pltpu.sync_copy(wf_hbm.at[pl.ds(wid * EPT, EPT)], w_v)

    def _zero(r, carry):
        for k in range(DD // 16):
            rows0[r, pl.ds(k * 16, 16)] = jnp.zeros((16,), jnp.float32)
        return carry

    lax.fori_loop(0, CH, _zero, 0)
    for t in range(ZSL // CH):
        pltpu.sync_copy(rows0, acc_sh.at[pl.ds(s * ZSL + t * CH, CH)])
    pltpu.sync_copy(rows0.at[pl.ds(0, ZSL % CH)],
                    acc_sh.at[pl.ds(s * ZSL + (ZSL // CH) * CH, ZSL % CH)])

    def _start_gather(j, buf, sem):
        pltpu.async_copy(hp_hbm.at[src_v.at[pl.ds(j * CH, CH)]], buf, sem)

    def _wait_gather(j, buf, sem):
        pltpu.make_async_copy(hp_hbm.at[src_v.at[pl.ds(j * CH, CH)]],
                              buf, sem).wait()

    def _start_scatter(j, buf, sem):
        pltpu.async_copy(buf, acc_sh.at[dst_v.at[pl.ds(j * CH, CH)]], sem,
                         add=True)

    def _wait_scatter(j, buf, sem):
        pltpu.make_async_copy(buf, acc_sh.at[dst_v.at[pl.ds(j * CH, CH)]],
                              sem).wait()

    def _scale(j, buf):
        def _body(i, c2):
            wb = plsc.load_gather(w_v, [jnp.full((16,), j * CH + i, jnp.int32)])
            for k in range(DD // 16):
                buf[i, pl.ds(k * 16, 16)] = buf[i, pl.ds(k * 16, 16)] * wb
            return c2

        lax.fori_loop(0, CH, _body, 0)

    _start_gather(0, rows0, g0)
    _start_gather(1, rows1, g1)
    plsc.subcore_barrier()

    def _pair(k, carry):
        j0 = k * 2
        j1 = j0 + 1
        _wait_gather(j0, rows0, g0)
        _scale(j0, rows0)
        _start_scatter(j0, rows0, s0)
        _wait_gather(j1, rows1, g1)
        _scale(j1, rows1)
        _start_scatter(j1, rows1, s1)
        _wait_scatter(j0, rows0, s0)
        _start_gather(j0 + 2, rows0, g0)

        @pl.when(j1 + 2 < NCHUNK)
        def _():
            _wait_scatter(j1, rows1, s1)
            _start_gather(j1 + 2, rows1, g1)

        return carry

    lax.fori_loop(0, (NCHUNK - 1) // 2, _pair, 0)
    _wait_gather(NCHUNK - 1, rows0, g0)
    _scale(NCHUNK - 1, rows0)
    _start_scatter(NCHUNK - 1, rows0, s0)
    _wait_scatter(NCHUNK - 2, rows1, s1)
    _wait_scatter(NCHUNK - 1, rows0, s0)
    plsc.subcore_barrier()

    @pl.when(s < NS - 1)
    def _():
        pltpu.sync_copy(acc_sh.at[pl.ds(s * WR, WR)],
                        out_hbm.at[c, pl.ds(s * WR, WR)])

    @pl.when(s == NS - 1)
    def _():
        pltpu.sync_copy(acc_sh.at[pl.ds((NS - 1) * WR, WR_LAST)],
                        out_hbm.at[c, pl.ds((NS - 1) * WR, WR_LAST)])


BN = 1000


def _dinv_of(deg_blk):
    dsum = deg_blk[0] + deg_blk[1]
    safe = jnp.where(dsum > 0, dsum, 1.0)
    return jnp.where(dsum > 0, lax.rsqrt(safe), 0.0)


def _tc_a_body(deg_ref, x_ref, w_ref, hp_ref):
    dinv = _dinv_of(deg_ref[...])
    h = jnp.dot(x_ref[...], w_ref[...], preferred_element_type=jnp.float32)
    hp_ref[...] = h * dinv


def _tc_b_body(deg_ref, acc_ref, b_ref, w_ref, h0r_ref, h1p_ref):
    dinv = _dinv_of(deg_ref[...])
    a = acc_ref[0] + acc_ref[1]
    h0r = jnp.maximum(a * dinv + b_ref[...], 0.0)
    h0r_ref[...] = h0r
    h1p_ref[...] = jnp.dot(h0r, w_ref[...],
                           preferred_element_type=jnp.float32) * dinv


def _tc_c_body(deg_ref, acc_ref, b_ref, h_ref):
    dinv = _dinv_of(deg_ref[...])
    h_ref[...] = (acc_ref[0] + acc_ref[1]) * dinv + b_ref[...]


_deg_spec = pl.BlockSpec((NC, BN, 1), lambda i: (0, i, 0))
_row_spec = pl.BlockSpec((BN, DD), lambda i: (i, 0))
_acc_spec = pl.BlockSpec((NC, BN, DD), lambda i: (0, i, 0))
_mat_spec = pl.BlockSpec((DD, DD), lambda i: (0, 0))
_bias_spec = pl.BlockSpec((1, DD), lambda i: (0, 0))
_GRID = NN // BN

_tc_a = pl.pallas_call(
    _tc_a_body,
    grid=(_GRID,),
    in_specs=[_deg_spec, _row_spec, _mat_spec],
    out_specs=_row_spec,
    out_shape=jax.ShapeDtypeStruct((NN, DD), jnp.float32),
)

_tc_b = pl.pallas_call(
    _tc_b_body,
    grid=(_GRID,),
    in_specs=[_deg_spec, _acc_spec, _bias_spec, _mat_spec],
    out_specs=[_row_spec, _row_spec],
    out_shape=[jax.ShapeDtypeStruct((NN, DD), jnp.float32),
               jax.ShapeDtypeStruct((NN, DD), jnp.float32)],
)

_tc_c = pl.pallas_call(
    _tc_c_body,
    grid=(_GRID,),
    in_specs=[_deg_spec, _acc_spec, _bias_spec],
    out_specs=_row_spec,
    out_shape=jax.ShapeDtypeStruct((NN, DD), jnp.float32),
)


def kernel(x, edge_index, edge_attr, W0, b0, W1, b1):
    srcf = edge_index[0]
    dstf = edge_index[1]
    dst3 = dstf.reshape(NW, NCHUNK, CH)
    w3 = edge_attr.reshape(NW, NCHUNK, CH)

    deg_flat = _deg_pass(dst3, w3)
    deg2col = deg_flat.reshape(NC, NPAD)[:, :NN].reshape(NC, NN, 1)

    h0p = _tc_a(deg2col, x, W0)
    acc0 = _edge_pass(h0p, srcf, dstf, edge_attr)
    h0r, h1p = _tc_b(deg2col, acc0, b0.reshape(1, DD), W1)
    acc1 = _edge_pass(h1p, srcf, dstf, edge_attr)
    h1 = _tc_c(deg2col, acc1, b1.reshape(1, DD))
    return (h0r, h1)


# speedup vs baseline: 18.3678x; 1.0464x over previous
import functools

import jax
import jax.numpy as jnp
from jax import lax
from jax.experimental import pallas as pl
from jax.experimental.pallas import tpu as pltpu
from jax.experimental.pallas import tpu_sc as plsc

NN = 10000
EE = 320000
DD = 128

NC = 2
NS = 16
NW = NC * NS
EPT = EE // NW
CH = 80
NCHUNK = EPT // CH
NPAD = 10240
DSL = NPAD // NS
CHP = 40
NCHP = EPT // CHP
NB = 4
ZSL = NN // NS
WR = 632
WR_LAST = NN - (NS - 1) * WR

_MESH = plsc.VectorSubcoreMesh(core_axis_name="c", subcore_axis_name="s")
_SC_PARAMS = pltpu.CompilerParams(needs_layout_passes=False)


@functools.partial(
    pl.kernel,
    out_type=jax.ShapeDtypeStruct((NC * NPAD,), jnp.float32),
    mesh=_MESH,
    compiler_params=_SC_PARAMS,
    scratch_types=[
        pltpu.VMEM((NCHUNK, CH), jnp.int32),
        pltpu.VMEM((NCHUNK, CH), jnp.float32),
        pltpu.VMEM((DSL,), jnp.float32),
        pltpu.VMEM_SHARED((NPAD,), jnp.float32),
    ],
)
def _deg_pass(dst_hbm, w_hbm, out_hbm, dst_v, w_v, stage_v, deg_sh):
    c = lax.axis_index("c")
    s = lax.axis_index("s")
    wid = c * NS + s
    pltpu.sync_copy(dst_hbm.at[wid], dst_v)
    pltpu.sync_copy(w_hbm.at[wid], w_v)

    def _zero(i, carry):
        stage_v[pl.ds(i * 16, 16)] = jnp.zeros((16,), jnp.float32)
        return carry

    lax.fori_loop(0, DSL // 16, _zero, 0)
    pltpu.sync_copy(stage_v, deg_sh.at[pl.ds(s * DSL, DSL)])
    plsc.subcore_barrier()

    def _scatter(j, carry):
        pltpu.sync_copy(w_v.at[j], deg_sh.at[dst_v.at[j]], add=True)
        return carry

    lax.fori_loop(0, NCHUNK, _scatter, 0)
    plsc.subcore_barrier()
    pltpu.sync_copy(deg_sh.at[pl.ds(s * DSL, DSL)],
                    out_hbm.at[pl.ds(c * NPAD + s * DSL, DSL)])


@functools.partial(
    pl.kernel,
    out_type=jax.ShapeDtypeStruct((NC, NN, DD), jnp.float32),
    mesh=_MESH,
    compiler_params=_SC_PARAMS,
    scratch_types=[
        pltpu.VMEM((EPT,), jnp.int32),
        pltpu.VMEM((EPT,), jnp.int32),
        pltpu.VMEM((EPT,), jnp.float32),
        [pltpu.VMEM((CHP, DD), jnp.float32) for _ in range(NB)],
        [pltpu.SemaphoreType.DMA for _ in range(NB)],
        [pltpu.SemaphoreType.DMA for _ in range(NB)],
        pltpu.VMEM_SHARED((NN, DD), jnp.float32),
    ],
)
def _edge_pass(hp_hbm, srcf_hbm, dstf_hbm, wf_hbm, out_hbm,
               src_v, dst_v, w_v, bufs, gsems, ssems, acc_sh):
    c = lax.axis_index("c")
    s = lax.axis_index("s")
    wid = c * NS + s
    pltpu.sync_copy(srcf_hbm.at[pl.ds(wid * EPT, EPT)], src_v)
    pltpu.sync_copy(dstf_hbm.at[pl.ds(wid * EPT, EPT)], dst_v)
    pltpu.sync_copy(wf_hbm.at[pl.ds(wid * EPT, EPT)], w_v)

    def _zero(r, carry):
        for k in range(DD // 16):
            bufs[0][r, pl.ds(k * 16, 16)] = jnp.zeros((16,), jnp.float32)
        return carry

    lax.fori_loop(0, CHP, _zero, 0)
    for t in range(ZSL // CHP):
        pltpu.sync_copy(bufs[0], acc_sh.at[pl.ds(s * ZSL + t * CHP, CHP)])
    pltpu.sync_copy(bufs[0].at[pl.ds(0, ZSL % CHP)],
                    acc_sh.at[pl.ds(s * ZSL + (ZSL // CHP) * CHP, ZSL % CHP)])

    def _start_gather(j, b):
        pltpu.async_copy(hp_hbm.at[src_v.at[pl.ds(j * CHP, CHP)]],
                         bufs[b], gsems[b])

    def _wait_gather(j, b):
        pltpu.make_async_copy(hp_hbm.at[src_v.at[pl.ds(j * CHP, CHP)]],
                              bufs[b], gsems[b]).wait()

    def _start_scatter(j, b):
        pltpu.async_copy(bufs[b], acc_sh.at[dst_v.at[pl.ds(j * CHP, CHP)]],
                         ssems[b], add=True)

    def _wait_scatter(j, b):
        pltpu.make_async_copy(bufs[b],
                              acc_sh.at[dst_v.at[pl.ds(j * CHP, CHP)]],
                              ssems[b]).wait()

    def _scale(j, b):
        buf = bufs[b]

        def _body(i, c2):
            wb = plsc.load_gather(w_v, [jnp.full((16,), j * CHP + i,
                                                 jnp.int32)])
            for k in range(DD // 16):
                buf[i, pl.ds(k * 16, 16)] = buf[i, pl.ds(k * 16, 16)] * wb
            return c2

        lax.fori_loop(0, CHP, _body, 0)

    _start_gather(0, 0)
    _start_gather(1, 1)
    plsc.subcore_barrier()

    def _step(j, b):
        _wait_gather(j, b)
        _scale(j, b)
        _start_scatter(j, b)

    for b in range(NB):
        _step(b, b)
        if b >= 2:
            _wait_scatter(b - 2, b - 2)
        _start_gather(b + 2, (b + 2) % NB)

    def _main(k, carry):
        j0 = k * NB
        for b in range(NB):
            j = j0 + b
            _step(j, b)
            _wait_scatter(j - 2, (b + 2) % NB)
            _start_gather(j + 2, (b + 2) % NB)
        return carry

    lax.fori_loop(1, NCHP // NB - 1, _main, 0)
    jlast = NCHP - 6
    for t in range(NB):
        j = jlast + t
        b = j % NB
        _step(j, b)
        _wait_scatter(j - 2, (j - 2) % NB)
        if j + 2 < NCHP:
            _start_gather(j + 2, (j + 2) % NB)
    for t in range(2):
        j = NCHP - 2 + t
        b = j % NB
        _step(j, b)
        _wait_scatter(j - 2, (j - 2) % NB)
    _wait_scatter(NCHP - 2, (NCHP - 2) % NB)
    _wait_scatter(NCHP - 1, (NCHP - 1) % NB)
    plsc.subcore_barrier()

    @pl.when(s < NS - 1)
    def _():
        pltpu.sync_copy(acc_sh.at[pl.ds(s * WR, WR)],
                        out_hbm.at[c, pl.ds(s * WR, WR)])

    @pl.when(s == NS - 1)
    def _():
        pltpu.sync_copy(acc_sh.at[pl.ds((NS - 1) * WR, WR_LAST)],
                        out_hbm.at[c, pl.ds((NS - 1) * WR, WR_LAST)])


BN = 1000


def _dinv_of(deg_blk):
    dsum = deg_blk[0] + deg_blk[1]
    safe = jnp.where(dsum > 0, dsum, 1.0)
    return jnp.where(dsum > 0, lax.rsqrt(safe), 0.0)


def _tc_a_body(deg_ref, x_ref, w_ref, hp_ref):
    dinv = _dinv_of(deg_ref[...])
    h = jnp.dot(x_ref[...], w_ref[...], preferred_element_type=jnp.float32)
    hp_ref[...] = h * dinv


def _tc_b_body(deg_ref, acc_ref, b_ref, w_ref, h0r_ref, h1p_ref):
    dinv = _dinv_of(deg_ref[...])
    a = acc_ref[0] + acc_ref[1]
    h0r = jnp.maximum(a * dinv + b_ref[...], 0.0)
    h0r_ref[...] = h0r
    h1p_ref[...] = jnp.dot(h0r, w_ref[...],
                           preferred_element_type=jnp.float32) * dinv


def _tc_c_body(deg_ref, acc_ref, b_ref, h_ref):
    dinv = _dinv_of(deg_ref[...])
    h_ref[...] = (acc_ref[0] + acc_ref[1]) * dinv + b_ref[...]


_deg_spec = pl.BlockSpec((NC, BN, 1), lambda i: (0, i, 0))
_row_spec = pl.BlockSpec((BN, DD), lambda i: (i, 0))
_acc_spec = pl.BlockSpec((NC, BN, DD), lambda i: (0, i, 0))
_mat_spec = pl.BlockSpec((DD, DD), lambda i: (0, 0))
_bias_spec = pl.BlockSpec((1, DD), lambda i: (0, 0))
_GRID = NN // BN

_tc_a = pl.pallas_call(
    _tc_a_body,
    grid=(_GRID,),
    in_specs=[_deg_spec, _row_spec, _mat_spec],
    out_specs=_row_spec,
    out_shape=jax.ShapeDtypeStruct((NN, DD), jnp.float32),
)

_tc_b = pl.pallas_call(
    _tc_b_body,
    grid=(_GRID,),
    in_specs=[_deg_spec, _acc_spec, _bias_spec, _mat_spec],
    out_specs=[_row_spec, _row_spec],
    out_shape=[jax.ShapeDtypeStruct((NN, DD), jnp.float32),
               jax.ShapeDtypeStruct((NN, DD), jnp.float32)],
)

_tc_c = pl.pallas_call(
    _tc_c_body,
    grid=(_GRID,),
    in_specs=[_deg_spec, _acc_spec, _bias_spec],
    out_specs=_row_spec,
    out_shape=jax.ShapeDtypeStruct((NN, DD), jnp.float32),
)


def kernel(x, edge_index, edge_attr, W0, b0, W1, b1):
    srcf = edge_index[0]
    dstf = edge_index[1]
    dst3 = dstf.reshape(NW, NCHUNK, CH)
    w3 = edge_attr.reshape(NW, NCHUNK, CH)

    deg_flat = _deg_pass(dst3, w3)
    deg2col = deg_flat.reshape(NC, NPAD)[:, :NN].reshape(NC, NN, 1)

    h0p = _tc_a(deg2col, x, W0)
    acc0 = _edge_pass(h0p, srcf, dstf, edge_attr)
    h0r, h1p = _tc_b(deg2col, acc0, b0.reshape(1, DD), W1)
    acc1 = _edge_pass(h1p, srcf, dstf, edge_attr)
    h1 = _tc_c(deg2col, acc1, b1.reshape(1, DD))
    return (h0r, h1)


# speedup vs baseline: 20.3904x; 1.1101x over previous
import functools

import jax
import jax.numpy as jnp
from jax import lax
from jax.experimental import pallas as pl
from jax.experimental.pallas import tpu as pltpu
from jax.experimental.pallas import tpu_sc as plsc

NN = 10000
EE = 320000
DD = 128

NC = 2
NS = 16
NW = NC * NS
EPT = EE // NW
CH = 80
NCHUNK = EPT // CH
NPAD = 10240
DSL = NPAD // NS
CHP = 40
NCHP = EPT // CHP
NB = 4
ZSL = NN // NS
WR = 632
WR_LAST = NN - (NS - 1) * WR

_MESH = plsc.VectorSubcoreMesh(core_axis_name="c", subcore_axis_name="s")
_SC_PARAMS = pltpu.CompilerParams(needs_layout_passes=False)


@functools.partial(
    pl.kernel,
    out_type=jax.ShapeDtypeStruct((NC * NPAD,), jnp.float32),
    mesh=_MESH,
    compiler_params=_SC_PARAMS,
    scratch_types=[
        pltpu.VMEM((NCHUNK, CH), jnp.int32),
        pltpu.VMEM((NCHUNK, CH), jnp.float32),
        pltpu.VMEM((DSL,), jnp.float32),
        pltpu.VMEM_SHARED((NPAD,), jnp.float32),
    ],
)
def _deg_pass(dst_hbm, w_hbm, out_hbm, dst_v, w_v, stage_v, deg_sh):
    c = lax.axis_index("c")
    s = lax.axis_index("s")
    wid = c * NS + s
    pltpu.sync_copy(dst_hbm.at[wid], dst_v)
    pltpu.sync_copy(w_hbm.at[wid], w_v)

    def _zero(i, carry):
        stage_v[pl.ds(i * 16, 16)] = jnp.zeros((16,), jnp.float32)
        return carry

    lax.fori_loop(0, DSL // 16, _zero, 0)
    pltpu.sync_copy(stage_v, deg_sh.at[pl.ds(s * DSL, DSL)])
    plsc.subcore_barrier()

    def _scatter(j, carry):
        pltpu.sync_copy(w_v.at[j], deg_sh.at[dst_v.at[j]], add=True)
        return carry

    lax.fori_loop(0, NCHUNK, _scatter, 0)
    plsc.subcore_barrier()
    pltpu.sync_copy(deg_sh.at[pl.ds(s * DSL, DSL)],
                    out_hbm.at[pl.ds(c * NPAD + s * DSL, DSL)])


@functools.partial(
    pl.kernel,
    out_type=jax.ShapeDtypeStruct((NC, NN, DD), jnp.float32),
    mesh=_MESH,
    compiler_params=_SC_PARAMS,
    scratch_types=[
        pltpu.VMEM((EPT,), jnp.int32),
        pltpu.VMEM((EPT,), jnp.int32),
        pltpu.VMEM((EPT,), jnp.float32),
        [pltpu.VMEM((CHP, DD), jnp.float32) for _ in range(NB)],
        [pltpu.SemaphoreType.DMA for _ in range(NB)],
        [pltpu.SemaphoreType.DMA for _ in range(NB)],
        pltpu.VMEM_SHARED((NN, DD), jnp.float32),
    ],
)
def _edge_pass(hp_hbm, srcf_hbm, dstf_hbm, wf_hbm, out_hbm,
               src_v, dst_v, w_v, bufs, gsems, ssems, acc_sh):
    c = lax.axis_index("c")
    s = lax.axis_index("s")
    wid = c * NS + s
    pltpu.sync_copy(srcf_hbm.at[pl.ds(wid * EPT, EPT)], src_v)
    pltpu.sync_copy(dstf_hbm.at[pl.ds(wid * EPT, EPT)], dst_v)
    pltpu.sync_copy(wf_hbm.at[pl.ds(wid * EPT, EPT)], w_v)

    def _zero(r, carry):
        for k in range(DD // 16):
            bufs[0][r, pl.ds(k * 16, 16)] = jnp.zeros((16,), jnp.float32)
        return carry

    lax.fori_loop(0, CHP, _zero, 0)
    for t in range(ZSL // CHP):
        pltpu.sync_copy(bufs[0], acc_sh.at[pl.ds(s * ZSL + t * CHP, CHP)])
    pltpu.sync_copy(bufs[0].at[pl.ds(0, ZSL % CHP)],
                    acc_sh.at[pl.ds(s * ZSL + (ZSL // CHP) * CHP, ZSL % CHP)])

    def _start_gather(j, b):
        pltpu.async_copy(hp_hbm.at[src_v.at[pl.ds(j * CHP, CHP)]],
                         bufs[b], gsems[b])

    def _wait_gather(j, b):
        pltpu.make_async_copy(hp_hbm.at[src_v.at[pl.ds(j * CHP, CHP)]],
                              bufs[b], gsems[b]).wait()

    def _start_scatter(j, b):
        pltpu.async_copy(bufs[b], acc_sh.at[dst_v.at[pl.ds(j * CHP, CHP)]],
                         ssems[b], add=True)

    def _wait_scatter(j, b):
        pltpu.make_async_copy(bufs[b],
                              acc_sh.at[dst_v.at[pl.ds(j * CHP, CHP)]],
                              ssems[b]).wait()

    def _scale(j, b):
        buf = bufs[b]

        @plsc.parallel_loop(0, CHP, unroll=4)
        def _body(i):
            wb = plsc.load_gather(w_v, [jnp.full((16,), j * CHP + i,
                                                 jnp.int32)])
            for k in range(DD // 16):
                buf[i, pl.ds(k * 16, 16)] = buf[i, pl.ds(k * 16, 16)] * wb

    _start_gather(0, 0)
    _start_gather(1, 1)
    plsc.subcore_barrier()

    def _step(j, b):
        _wait_gather(j, b)
        _scale(j, b)
        _start_scatter(j, b)

    for b in range(NB):
        _step(b, b)
        if b >= 2:
            _wait_scatter(b - 2, b - 2)
        _start_gather(b + 2, (b + 2) % NB)

    def _main(k, carry):
        j0 = k * NB
        for b in range(NB):
            j = j0 + b
            _step(j, b)
            _wait_scatter(j - 2, (b + 2) % NB)
            _start_gather(j + 2, (b + 2) % NB)
        return carry

    lax.fori_loop(1, NCHP // NB - 1, _main, 0)
    jlast = NCHP - 6
    for t in range(NB):
        j = jlast + t
        b = j % NB
        _step(j, b)
        _wait_scatter(j - 2, (j - 2) % NB)
        if j + 2 < NCHP:
            _start_gather(j + 2, (j + 2) % NB)
    for t in range(2):
        j = NCHP - 2 + t
        b = j % NB
        _step(j, b)
        _wait_scatter(j - 2, (j - 2) % NB)
    _wait_scatter(NCHP - 2, (NCHP - 2) % NB)
    _wait_scatter(NCHP - 1, (NCHP - 1) % NB)
    plsc.subcore_barrier()

    @pl.when(s < NS - 1)
    def _():
        pltpu.sync_copy(acc_sh.at[pl.ds(s * WR, WR)],
                        out_hbm.at[c, pl.ds(s * WR, WR)])

    @pl.when(s == NS - 1)
    def _():
        pltpu.sync_copy(acc_sh.at[pl.ds((NS - 1) * WR, WR_LAST)],
                        out_hbm.at[c, pl.ds((NS - 1) * WR, WR_LAST)])


BN = 1000


def _dinv_of(deg_blk):
    dsum = deg_blk[0] + deg_blk[1]
    safe = jnp.where(dsum > 0, dsum, 1.0)
    return jnp.where(dsum > 0, lax.rsqrt(safe), 0.0)


def _tc_a_body(deg_ref, x_ref, w_ref, hp_ref):
    dinv = _dinv_of(deg_ref[...])
    h = jnp.dot(x_ref[...], w_ref[...], preferred_element_type=jnp.float32)
    hp_ref[...] = h * dinv


def _tc_b_body(deg_ref, acc_ref, b_ref, w_ref, h0r_ref, h1p_ref):
    dinv = _dinv_of(deg_ref[...])
    a = acc_ref[0] + acc_ref[1]
    h0r = jnp.maximum(a * dinv + b_ref[...], 0.0)
    h0r_ref[...] = h0r
    h1p_ref[...] = jnp.dot(h0r, w_ref[...],
                           preferred_element_type=jnp.float32) * dinv


def _tc_c_body(deg_ref, acc_ref, b_ref, h_ref):
    dinv = _dinv_of(deg_ref[...])
    h_ref[...] = (acc_ref[0] + acc_ref[1]) * dinv + b_ref[...]


_deg_spec = pl.BlockSpec((NC, BN, 1), lambda i: (0, i, 0))
_row_spec = pl.BlockSpec((BN, DD), lambda i: (i, 0))
_acc_spec = pl.BlockSpec((NC, BN, DD), lambda i: (0, i, 0))
_mat_spec = pl.BlockSpec((DD, DD), lambda i: (0, 0))
_bias_spec = pl.BlockSpec((1, DD), lambda i: (0, 0))
_GRID = NN // BN

_tc_a = pl.pallas_call(
    _tc_a_body,
    grid=(_GRID,),
    in_specs=[_deg_spec, _row_spec, _mat_spec],
    out_specs=_row_spec,
    out_shape=jax.ShapeDtypeStruct((NN, DD), jnp.float32),
)

_tc_b = pl.pallas_call(
    _tc_b_body,
    grid=(_GRID,),
    in_specs=[_deg_spec, _acc_spec, _bias_spec, _mat_spec],
    out_specs=[_row_spec, _row_spec],
    out_shape=[jax.ShapeDtypeStruct((NN, DD), jnp.float32),
               jax.ShapeDtypeStruct((NN, DD), jnp.float32)],
)

_tc_c = pl.pallas_call(
    _tc_c_body,
    grid=(_GRID,),
    in_specs=[_deg_spec, _acc_spec, _bias_spec],
    out_specs=_row_spec,
    out_shape=jax.ShapeDtypeStruct((NN, DD), jnp.float32),
)


def kernel(x, edge_index, edge_attr, W0, b0, W1, b1):
    srcf = edge_index[0]
    dstf = edge_index[1]
    dst3 = dstf.reshape(NW, NCHUNK, CH)
    w3 = edge_attr.reshape(NW, NCHUNK, CH)

    deg_flat = _deg_pass(dst3, w3)
    deg2col = deg_flat.reshape(NC, NPAD)[:, :NN].reshape(NC, NN, 1)

    h0p = _tc_a(deg2col, x, W0)
    acc0 = _edge_pass(h0p, srcf, dstf, edge_attr)
    h0r, h1p = _tc_b(deg2col, acc0, b0.reshape(1, DD), W1)
    acc1 = _edge_pass(h1p, srcf, dstf, edge_attr)
    h1 = _tc_c(deg2col, acc1, b1.reshape(1, DD))
    return (h0r, h1)


# speedup vs baseline: 26.4287x; 1.2961x over previous
import functools

import jax
import jax.numpy as jnp
import numpy as np
from jax import lax
from jax.experimental import pallas as pl
from jax.experimental.pallas import tpu as pltpu
from jax.experimental.pallas import tpu_sc as plsc

NN = 10000
EE = 320000
DD = 128

NC = 2
NS = 16
NW = NC * NS
EPT = EE // NW
CH = 80
NCHUNK = EPT // CH
NPAD = 10240
DSL = NPAD // NS
CHP = 40
NCHP = EPT // CHP
NB = 4
NFB = 2

_PERM = np.arange(DD).reshape(DD // 32, 2, 16).transpose(0, 2, 1).reshape(DD)
ZSL = NN // NS
WR = 632
WR_LAST = NN - (NS - 1) * WR

_MESH = plsc.VectorSubcoreMesh(core_axis_name="c", subcore_axis_name="s")
_SC_PARAMS = pltpu.CompilerParams(needs_layout_passes=False)
_SC_PARAMS_NT = pltpu.CompilerParams(needs_layout_passes=False,
                                     use_tc_tiling_on_sc=False)


@functools.partial(
    pl.kernel,
    out_type=jax.ShapeDtypeStruct((NC * NPAD,), jnp.float32),
    mesh=_MESH,
    compiler_params=_SC_PARAMS,
    scratch_types=[
        pltpu.VMEM((NCHUNK, CH), jnp.int32),
        pltpu.VMEM((NCHUNK, CH), jnp.float32),
        pltpu.VMEM((DSL,), jnp.float32),
        pltpu.VMEM_SHARED((NPAD,), jnp.float32),
    ],
)
def _deg_pass(dst_hbm, w_hbm, out_hbm, dst_v, w_v, stage_v, deg_sh):
    c = lax.axis_index("c")
    s = lax.axis_index("s")
    wid = c * NS + s
    pltpu.sync_copy(dst_hbm.at[wid], dst_v)
    pltpu.sync_copy(w_hbm.at[wid], w_v)

    def _zero(i, carry):
        stage_v[pl.ds(i * 16, 16)] = jnp.zeros((16,), jnp.float32)
        return carry

    lax.fori_loop(0, DSL // 16, _zero, 0)
    pltpu.sync_copy(stage_v, deg_sh.at[pl.ds(s * DSL, DSL)])
    plsc.subcore_barrier()

    def _scatter(j, carry):
        pltpu.sync_copy(w_v.at[j], deg_sh.at[dst_v.at[j]], add=True)
        return carry

    lax.fori_loop(0, NCHUNK, _scatter, 0)
    plsc.subcore_barrier()
    pltpu.sync_copy(deg_sh.at[pl.ds(s * DSL, DSL)],
                    out_hbm.at[pl.ds(c * NPAD + s * DSL, DSL)])


@functools.partial(
    pl.kernel,
    out_type=jax.ShapeDtypeStruct((NC, NN, DD), jnp.float32),
    mesh=_MESH,
    compiler_params=_SC_PARAMS_NT,
    scratch_types=[
        pltpu.VMEM((EPT,), jnp.int32),
        pltpu.VMEM((EPT,), jnp.int32),
        pltpu.VMEM((EPT,), jnp.float32),
        [pltpu.VMEM((CHP, DD), jnp.bfloat16) for _ in range(NB)],
        [pltpu.VMEM((CHP, DD), jnp.float32) for _ in range(NFB)],
        [pltpu.SemaphoreType.DMA for _ in range(NB)],
        [pltpu.SemaphoreType.DMA for _ in range(NFB)],
        pltpu.VMEM_SHARED((NN, DD), jnp.float32),
    ],
)
def _edge_pass(hp_hbm, srcf_hbm, dstf_hbm, wf_hbm, out_hbm,
               src_v, dst_v, w_v, bbufs, fbufs, gsems, ssems, acc_sh):
    c = lax.axis_index("c")
    s = lax.axis_index("s")
    wid = c * NS + s
    pltpu.sync_copy(srcf_hbm.at[pl.ds(wid * EPT, EPT)], src_v)
    pltpu.sync_copy(dstf_hbm.at[pl.ds(wid * EPT, EPT)], dst_v)
    pltpu.sync_copy(wf_hbm.at[pl.ds(wid * EPT, EPT)], w_v)

    def _zero(r, carry):
        for k in range(DD // 16):
            fbufs[0][r, pl.ds(k * 16, 16)] = jnp.zeros((16,), jnp.float32)
        return carry

    lax.fori_loop(0, CHP, _zero, 0)
    for t in range(ZSL // CHP):
        pltpu.sync_copy(fbufs[0], acc_sh.at[pl.ds(s * ZSL + t * CHP, CHP)])
    pltpu.sync_copy(fbufs[0].at[pl.ds(0, ZSL % CHP)],
                    acc_sh.at[pl.ds(s * ZSL + (ZSL // CHP) * CHP, ZSL % CHP)])

    def _start_gather(j, b):
        pltpu.async_copy(hp_hbm.at[src_v.at[pl.ds(j * CHP, CHP)]],
                         bbufs[b], gsems[b])

    def _wait_gather(j, b):
        pltpu.make_async_copy(hp_hbm.at[src_v.at[pl.ds(j * CHP, CHP)]],
                              bbufs[b], gsems[b]).wait()

    def _start_scatter(j, fb):
        pltpu.async_copy(fbufs[fb], acc_sh.at[dst_v.at[pl.ds(j * CHP, CHP)]],
                         ssems[fb], add=True)

    def _wait_scatter(j, fb):
        pltpu.make_async_copy(fbufs[fb],
                              acc_sh.at[dst_v.at[pl.ds(j * CHP, CHP)]],
                              ssems[fb]).wait()

    def _scale(j, b, fb):
        bbuf = bbufs[b]
        fbuf = fbufs[fb]

        @plsc.parallel_loop(0, CHP, unroll=4)
        def _body(i):
            wb = plsc.load_gather(w_v, [jnp.full((16,), j * CHP + i,
                                                 jnp.int32)])
            for k in range(DD // 32):
                ab = bbuf[i, pl.ds(k * 32, 32)]
                lo, hi = plsc.unpack(ab, format=plsc.PackFormat.INTERLEAVED)
                fbuf[i, pl.ds(k * 32, 16)] = lo * wb
                fbuf[i, pl.ds(k * 32 + 16, 16)] = hi * wb

    for b in range(NB):
        _start_gather(b, b)
    plsc.subcore_barrier()

    for b in range(NB):
        _wait_gather(b, b)
        if b >= NFB:
            _wait_scatter(b - NFB, b % NFB)
        _scale(b, b, b % NFB)
        _start_gather(b + NB, b)
        _start_scatter(b, b % NFB)

    def _main(k, carry):
        j0 = k * NB
        for b in range(NB):
            j = j0 + b
            fb = b % NFB
            _wait_gather(j, b)
            _wait_scatter(j - NFB, fb)
            _scale(j, b, fb)

            @pl.when(j + NB < NCHP)
            def _():
                _start_gather(j + NB, b)

            _start_scatter(j, fb)
        return carry

    lax.fori_loop(1, NCHP // NB, _main, 0)
    for t in range(NCHP % NB):
        j = (NCHP // NB) * NB + t
        b = j % NB
        fb = b % NFB
        _wait_gather(j, b)
        _wait_scatter(j - NFB, fb)
        _scale(j, b, fb)
        _start_scatter(j, fb)
    _wait_scatter(NCHP - 2, ((NCHP - 2) % NB) % NFB)
    _wait_scatter(NCHP - 1, ((NCHP - 1) % NB) % NFB)
    plsc.subcore_barrier()

    @pl.when(s < NS - 1)
    def _():
        pltpu.sync_copy(acc_sh.at[pl.ds(s * WR, WR)],
                        out_hbm.at[c, pl.ds(s * WR, WR)])

    @pl.when(s == NS - 1)
    def _():
        pltpu.sync_copy(acc_sh.at[pl.ds((NS - 1) * WR, WR_LAST)],
                        out_hbm.at[c, pl.ds((NS - 1) * WR, WR_LAST)])


BN = 1000


def _dinv_of(deg_blk):
    dsum = deg_blk[0] + deg_blk[1]
    safe = jnp.where(dsum > 0, dsum, 1.0)
    return jnp.where(dsum > 0, lax.rsqrt(safe), 0.0)


def _tc_a_body(deg_ref, x_ref, w_ref, hp_ref):
    dinv = _dinv_of(deg_ref[...])
    h = jnp.dot(x_ref[...], w_ref[...], preferred_element_type=jnp.float32)
    hp_ref[...] = (h * dinv).astype(jnp.bfloat16)


def _tc_b_body(deg_ref, acc_ref, b_ref, w_ref, h0r_ref, h1p_ref):
    dinv = _dinv_of(deg_ref[...])
    a = acc_ref[0] + acc_ref[1]
    h0r = jnp.maximum(a * dinv + b_ref[...], 0.0)
    h0r_ref[...] = h0r
    h1p_ref[...] = (jnp.dot(h0r, w_ref[...],
                            preferred_element_type=jnp.float32)
                    * dinv).astype(jnp.bfloat16)


def _tc_c_body(deg_ref, acc_ref, b_ref, h_ref):
    dinv = _dinv_of(deg_ref[...])
    h_ref[...] = (acc_ref[0] + acc_ref[1]) * dinv + b_ref[...]


_deg_spec = pl.BlockSpec((NC, BN, 1), lambda i: (0, i, 0))
_row_spec = pl.BlockSpec((BN, DD), lambda i: (i, 0))
_acc_spec = pl.BlockSpec((NC, BN, DD), lambda i: (0, i, 0))
_mat_spec = pl.BlockSpec((DD, DD), lambda i: (0, 0))
_bias_spec = pl.BlockSpec((1, DD), lambda i: (0, 0))
_GRID = NN // BN

_tc_a = pl.pallas_call(
    _tc_a_body,
    grid=(_GRID,),
    in_specs=[_deg_spec, _row_spec, _mat_spec],
    out_specs=_row_spec,
    out_shape=jax.ShapeDtypeStruct((NN, DD), jnp.bfloat16),
)

_tc_b = pl.pallas_call(
    _tc_b_body,
    grid=(_GRID,),
    in_specs=[_deg_spec, _acc_spec, _bias_spec, _mat_spec],
    out_specs=[_row_spec, _row_spec],
    out_shape=[jax.ShapeDtypeStruct((NN, DD), jnp.float32),
               jax.ShapeDtypeStruct((NN, DD), jnp.bfloat16)],
)

_tc_c = pl.pallas_call(
    _tc_c_body,
    grid=(_GRID,),
    in_specs=[_deg_spec, _acc_spec, _bias_spec],
    out_specs=_row_spec,
    out_shape=jax.ShapeDtypeStruct((NN, DD), jnp.float32),
)


def kernel(x, edge_index, edge_attr, W0, b0, W1, b1):
    srcf = edge_index[0]
    dstf = edge_index[1]
    dst3 = dstf.reshape(NW, NCHUNK, CH)
    w3 = edge_attr.reshape(NW, NCHUNK, CH)

    deg_flat = _deg_pass(dst3, w3)
    deg2col = deg_flat.reshape(NC, NPAD)[:, :NN].reshape(NC, NN, 1)

    h0p = _tc_a(deg2col, x, W0[:, _PERM])
    acc0 = _edge_pass(h0p, srcf, dstf, edge_attr)
    h0r, h1p = _tc_b(deg2col, acc0, b0.reshape(1, DD), W1[:, _PERM])
    acc1 = _edge_pass(h1p, srcf, dstf, edge_attr)
    h1 = _tc_c(deg2col, acc1, b1.reshape(1, DD))
    return (h0r, h1)
